# baseline (speedup 1.0000x reference)
import functools

import jax
import jax.numpy as jnp
from jax import lax
from jax.experimental import pallas as pl
from jax.experimental.pallas import tpu as pltpu
from jax.experimental.pallas import tpu_sc as plsc

D_MODEL = 1024
MAX_LEN = 8192
BATCH = 16384

_NC = 2
_NS = 16
_NW = _NC * _NS

_B_PER_W = BATCH // _NW
_C = 16
_NCH = _B_PER_W // _C
_NBUF = 3


def _make_gather():
    mesh = plsc.VectorSubcoreMesh(core_axis_name="c", subcore_axis_name="s")

    @functools.partial(
        pl.kernel,
        mesh=mesh,
        out_type=jax.ShapeDtypeStruct((BATCH, D_MODEL), jnp.float32),
        scratch_types=[
            pltpu.VMEM((_NCH, _C), jnp.int32),
            pltpu.VMEM((_NBUF, _C, D_MODEL), jnp.float32),
            pltpu.VMEM_SHARED((_NS, _NBUF, _C, D_MODEL), jnp.float32),
        ] + [pltpu.SemaphoreType.DMA] * (3 * _NBUF),
    )
    def gather_kernel(table_hbm, idx_hbm, out_hbm, idx_v, rows_v, rows_sh,
                      *sems):
        sid = lax.axis_index("s")
        wid = sid * _NC + lax.axis_index("c")
        base = wid * _B_PER_W
        my_sh = rows_sh.at[sid]
        pltpu.sync_copy(idx_hbm.at[wid], idx_v)

        gsems = sems[:_NBUF]
        xsems = sems[_NBUF:2 * _NBUF]
        ssems = sems[2 * _NBUF:]

        g = {}
        x = {}
        s = {}
        for j in range(_NCH + 2):
            if j < _NCH:
                b = j % _NBUF
                if j - _NBUF in x:
                    x.pop(j - _NBUF).wait()
                g[j] = pltpu.async_copy(
                    table_hbm.at[idx_v.at[j]], rows_v.at[b], gsems[b])
            jj = j - 1
            if 0 <= jj < _NCH:
                b = jj % _NBUF
                if jj - _NBUF in s:
                    s.pop(jj - _NBUF).wait()
                g.pop(jj).wait()
                x[jj] = pltpu.async_copy(rows_v.at[b], my_sh.at[b], xsems[b])
            jk = j - 2
            if 0 <= jk < _NCH:
                b = jk % _NBUF
                if jk in x:
                    x.pop(jk).wait()
                s[jk] = pltpu.async_copy(
                    my_sh.at[b], out_hbm.at[pl.ds(base + jk * _C, _C)],
                    ssems[b])
        for jk in sorted(s):
            s[jk].wait()

    return gather_kernel


_gather = _make_gather()


def kernel(pe, index):
    idx = index.astype(jnp.int32).reshape(_NW, _NCH, _C)
    return _gather(pe, idx)
